# baseline (speedup 1.0000x reference)
import functools

import jax
import jax.numpy as jnp
import numpy as np
from jax import lax
from jax.experimental import pallas as pl
from jax.experimental.pallas import tpu as pltpu
from jax.experimental.pallas import tpu_sc as plsc

jax.config.update("jax_enable_x64", True)

_H = 16
_TP = 8
_NUM_SC_WORKERS = 32
_TB = 128


def _sc_gather(tab_pl, idx_all):
    B = idx_all.shape[0]
    bpw = B // _NUM_SC_WORKERS
    mesh = plsc.VectorSubcoreMesh(core_axis_name="c", subcore_axis_name="s")

    @functools.partial(
        pl.kernel,
        mesh=mesh,
        compiler_params=pltpu.CompilerParams(use_tc_tiling_on_sc=False),
        out_type=jax.ShapeDtypeStruct((_TP, B), jnp.int32),
        scratch_types=[
            pltpu.VMEM((bpw,), jnp.int32),
            pltpu.VMEM((_TP, bpw), jnp.int32),
            pltpu.SemaphoreType.DMA,
        ],
    )
    def gather_kernel(tab_hbm, idx_hbm, out_hbm, idx_v, rows_v, sem):
        wid = lax.axis_index("s") * 2 + lax.axis_index("c")
        base = wid * bpw
        pltpu.sync_copy(idx_hbm.at[pl.ds(base, bpw)], idx_v)
        handles = [pltpu.async_copy(tab_hbm.at[jnp.int32(p)].at[idx_v],
                                    rows_v.at[jnp.int32(p)], sem)
                   for p in range(_TP)]
        for h in handles:
            h.wait()
        pltpu.sync_copy(rows_v, out_hbm.at[:, pl.ds(base, bpw)])

    return gather_kernel(tab_pl, idx_all)


def _tc_body(locs_ref, rmask_ref, semb_ref, nemb_ref, mask_ref, sel_ref, rl_ref):
    locs = locs_ref[...]
    sta = locs[0]
    pos = locs[1:]
    rm = rmask_ref[...]

    apos = jnp.abs(pos)
    pneg = pos < 0

    def cos16(cand):
        xr = apos ^ jnp.abs(cand)[None]
        f = (xr + 1).astype(jnp.float32)
        ex = lax.bitcast_convert_type(f, jnp.int32) >> 23
        mag = 142 - ex
        sgneg = pneg ^ (cand < 0)[None]
        return jnp.where(sgneg, -mag, mag)

    cs16 = cos16(sta)
    s16 = jnp.sum(cs16, axis=1, dtype=jnp.int32)
    base16 = s16[:, None, :] - cs16

    semb = semb_ref[...]
    nemb = nemb_ref[...]
    msk = mask_ref[...]
    sn = jnp.maximum(jnp.sqrt(jnp.sum(semb * semb, axis=0, keepdims=True)), 1e-12)
    nn = jnp.maximum(jnp.sqrt(jnp.sum(nemb * nemb, axis=1)), 1e-12)
    eu = jnp.sum(semb[None, :, :] * nemb, axis=1) / (sn * nn)
    w = jnp.abs(eu) * msk
    lth_inv = 1.0 / (jnp.sum(msk, axis=0, keepdims=True) + 1e-12)
    eu_b = eu[:, None, :]
    w_b = w[:, None, :]

    corr = jnp.log(jnp.float32(1000000.0) / jnp.float32(32.0))
    inv128 = np.float32(1.0 / 128.0)
    inv_tp = np.float32(1.0 / _TP)

    hidx = lax.broadcasted_iota(jnp.int32, (_H, 1, 1), 0)
    rmm = rm & (jnp.left_shift(jnp.int32(1), hidx) - 1)

    def body(c, carry):
        c = c.astype(jnp.int32)
        best_val, best_cos, best_cro, best_loc = carry
        j = jnp.where(c < _H, c, c - (_H + 1))
        j = jnp.maximum(j, 0)
        bit = jnp.left_shift(jnp.int32(1), j)
        rmj = jnp.sum(jnp.where(hidx == j, rmm, 0), axis=0,
                      dtype=jnp.int32)
        cand0 = (sta ^ bit) ^ rmj
        cand = jnp.where(c == _H, sta,
                         jnp.where(c < _H, cand0, -cand0))

        ct = (base16 + cos16(cand)).astype(jnp.float32) * inv128
        d = ct[:32] - eu_b
        lcos = jnp.sum(d * d * w_b, axis=0) * lth_inv

        x = ct[32:] * 20.0 + corr
        m = jnp.max(x, axis=0)
        lse = jnp.log(jnp.sum(jnp.exp(x - m[None]), axis=0)) + m
        lcro = lse - ct[32] * 20.0

        ltot = lcos + lcro
        upd = ltot < best_val
        return (jnp.where(upd, ltot, best_val),
                jnp.where(upd, lcos, best_cos),
                jnp.where(upd, lcro, best_cro),
                jnp.where(upd, cand, best_loc))

    tb = sta.shape[-1]
    init = (jnp.full((_TP, tb), jnp.inf, jnp.float32),
            jnp.zeros((_TP, tb), jnp.float32),
            jnp.zeros((_TP, tb), jnp.float32),
            jnp.zeros((_TP, tb), jnp.int32))
    best_val, best_cos, best_cro, best_loc = lax.fori_loop(
        jnp.int32(0), jnp.int32(2 * _H + 1), body, init, unroll=False)

    sel_ref[...] = best_loc
    rl = jnp.concatenate([
        jnp.sum(best_cos, axis=0, keepdims=True) * inv_tp,
        jnp.sum(best_cro, axis=0, keepdims=True) * inv_tp,
        jnp.sum(best_val, axis=0, keepdims=True) * inv_tp,
        jnp.zeros((5, tb), jnp.float32),
    ], axis=0)
    rl_ref[...] = rl


def _tc_compute(locs_t, rmask_t, semb_t, nemb_t, mask_t):
    t = locs_t.shape[-1]
    grid = (t // _TB,)
    sel, rl = pl.pallas_call(
        _tc_body,
        grid=grid,
        in_specs=[
            pl.BlockSpec((65, _TP, _TB), lambda i: (i * 0, i * 0, i)),
            pl.BlockSpec((_H, _TP, _TB), lambda i: (i * 0, i * 0, i)),
            pl.BlockSpec((64, _TB), lambda i: (i * 0, i)),
            pl.BlockSpec((32, 64, _TB), lambda i: (i * 0, i * 0, i)),
            pl.BlockSpec((32, _TB), lambda i: (i * 0, i)),
        ],
        out_specs=[
            pl.BlockSpec((_TP, _TB), lambda i: (i * 0, i)),
            pl.BlockSpec((8, _TB), lambda i: (i * 0, i)),
        ],
        out_shape=[
            jax.ShapeDtypeStruct((_TP, t), jnp.int32),
            jax.ShapeDtypeStruct((8, t), jnp.float32),
        ],
        compiler_params=pltpu.CompilerParams(
            dimension_semantics=("arbitrary",)),
    )(locs_t, rmask_t, semb_t, nemb_t, mask_t)
    return sel, rl


def kernel(sta_idx, nei_idx, voc_idx, sta_emb, nei_emb, voc_emb, random_masks,
           mask, main_locations):
    t = sta_idx.shape[0]
    n_nbr = nei_idx.shape[1]
    k_voc = voc_idx.shape[1]

    idx_all = jnp.concatenate(
        [sta_idx[:, None], nei_idx, voc_idx], axis=1).astype(jnp.int32)
    idx_flat = idx_all.reshape(t * (1 + n_nbr + k_voc))

    tab_pl = main_locations.T.astype(jnp.int32)

    gathered = _sc_gather(tab_pl, idx_flat)
    locs = gathered.reshape(_TP, t, 1 + n_nbr + k_voc)
    locs_t = locs.transpose(2, 0, 1)

    rmask_t = random_masks.astype(jnp.int32).reshape(t, _H, _TP).transpose(1, 2, 0)
    semb_t = sta_emb.T
    nemb_t = nei_emb.transpose(1, 2, 0)
    mask_t = mask.T

    sel, rl = _tc_compute(locs_t, rmask_t, semb_t, nemb_t, mask_t)

    selected_locs = sel.T.astype(jnp.int64)
    return selected_locs, rl[0], rl[1], rl[2]


# speedup vs baseline: 1.8774x; 1.8774x over previous
import functools

import jax
import jax.numpy as jnp
import numpy as np
from jax import lax
from jax.experimental import pallas as pl
from jax.experimental.pallas import tpu as pltpu
from jax.experimental.pallas import tpu_sc as plsc

jax.config.update("jax_enable_x64", True)

_H = 16
_TP = 8
_NUM_SC_WORKERS = 32
_TB = 128


def _sc_gather(planes, idx_all):
    B = idx_all.shape[0]
    bpw = B // _NUM_SC_WORKERS
    mesh = plsc.VectorSubcoreMesh(core_axis_name="c", subcore_axis_name="s")

    @functools.partial(
        pl.kernel,
        mesh=mesh,
        compiler_params=pltpu.CompilerParams(use_tc_tiling_on_sc=False),
        out_type=jax.ShapeDtypeStruct((_TP, B), jnp.int32),
        scratch_types=[
            pltpu.VMEM((bpw,), jnp.int32),
            pltpu.VMEM((_TP, bpw), jnp.int32),
            pltpu.SemaphoreType.DMA,
        ],
    )
    def gather_kernel(*refs):
        tabs = refs[:_TP]
        idx_hbm, out_hbm, idx_v, rows_v, sem = refs[_TP:]
        wid = lax.axis_index("s") * 2 + lax.axis_index("c")
        base = wid * bpw
        pltpu.sync_copy(idx_hbm.at[pl.ds(base, bpw)], idx_v)
        handles = [pltpu.async_copy(tabs[p].at[idx_v],
                                    rows_v.at[jnp.int32(p)], sem)
                   for p in range(_TP)]
        for h in handles:
            h.wait()
        pltpu.sync_copy(rows_v, out_hbm.at[:, pl.ds(base, bpw)])

    return gather_kernel(*planes, idx_all)


def _tc_body(locs_ref, rmask_ref, semb_ref, nemb_ref, mask_ref, sel_ref, rl_ref):
    locs = locs_ref[...]
    sta = locs[0]
    pos = locs[1:]
    rm = rmask_ref[...]

    apos = jnp.abs(pos)
    pneg = pos < 0

    def cos16(cand):
        xr = apos ^ jnp.abs(cand)[None]
        f = (xr + 1).astype(jnp.float32)
        ex = lax.bitcast_convert_type(f, jnp.int32) >> 23
        mag = 142 - ex
        sgneg = pneg ^ (cand < 0)[None]
        return jnp.where(sgneg, -mag, mag)

    cs16 = cos16(sta)
    s16 = jnp.sum(cs16, axis=1, dtype=jnp.int32)
    base16 = s16[:, None, :] - cs16

    semb = semb_ref[...]
    nemb = nemb_ref[...]
    msk = mask_ref[...]
    sn = jnp.maximum(jnp.sqrt(jnp.sum(semb * semb, axis=0, keepdims=True)), 1e-12)
    nn = jnp.maximum(jnp.sqrt(jnp.sum(nemb * nemb, axis=1)), 1e-12)
    eu = jnp.sum(semb[None, :, :] * nemb, axis=1) / (sn * nn)
    w = jnp.abs(eu) * msk
    lth_inv = 1.0 / (jnp.sum(msk, axis=0, keepdims=True) + 1e-12)
    eu_b = eu[:, None, :]
    w_b = w[:, None, :]

    corr = jnp.log(jnp.float32(1000000.0) / jnp.float32(32.0))
    inv128 = np.float32(1.0 / 128.0)
    inv_tp = np.float32(1.0 / _TP)

    hidx = lax.broadcasted_iota(jnp.int32, (_H, 1, 1), 0)
    rmm = rm & (jnp.left_shift(jnp.int32(1), hidx) - 1)

    def body(c, carry):
        c = c.astype(jnp.int32)
        best_val, best_cos, best_cro, best_loc = carry
        j = jnp.where(c < _H, c, c - (_H + 1))
        j = jnp.maximum(j, 0)
        bit = jnp.left_shift(jnp.int32(1), j)
        rmj = jnp.sum(jnp.where(hidx == j, rmm, 0), axis=0,
                      dtype=jnp.int32)
        cand0 = (sta ^ bit) ^ rmj
        cand = jnp.where(c == _H, sta,
                         jnp.where(c < _H, cand0, -cand0))

        ct = (base16 + cos16(cand)).astype(jnp.float32) * inv128
        d = ct[:32] - eu_b
        lcos = jnp.sum(d * d * w_b, axis=0) * lth_inv

        x = ct[32:] * 20.0 + corr
        m = jnp.max(x, axis=0)
        lse = jnp.log(jnp.sum(jnp.exp(x - m[None]), axis=0)) + m
        lcro = lse - ct[32] * 20.0

        ltot = lcos + lcro
        upd = ltot < best_val
        return (jnp.where(upd, ltot, best_val),
                jnp.where(upd, lcos, best_cos),
                jnp.where(upd, lcro, best_cro),
                jnp.where(upd, cand, best_loc))

    tb = sta.shape[-1]
    init = (jnp.full((_TP, tb), jnp.inf, jnp.float32),
            jnp.zeros((_TP, tb), jnp.float32),
            jnp.zeros((_TP, tb), jnp.float32),
            jnp.zeros((_TP, tb), jnp.int32))
    best_val, best_cos, best_cro, best_loc = lax.fori_loop(
        jnp.int32(0), jnp.int32(2 * _H + 1), body, init, unroll=False)

    sel_ref[...] = best_loc
    rl = jnp.concatenate([
        jnp.sum(best_cos, axis=0, keepdims=True) * inv_tp,
        jnp.sum(best_cro, axis=0, keepdims=True) * inv_tp,
        jnp.sum(best_val, axis=0, keepdims=True) * inv_tp,
        jnp.zeros((5, tb), jnp.float32),
    ], axis=0)
    rl_ref[...] = rl


def _tc_compute(locs_t, rmask_t, semb_t, nemb_t, mask_t):
    t = locs_t.shape[-1]
    grid = (t // _TB,)
    sel, rl = pl.pallas_call(
        _tc_body,
        grid=grid,
        in_specs=[
            pl.BlockSpec((65, _TP, _TB), lambda i: (i * 0, i * 0, i)),
            pl.BlockSpec((_H, _TP, _TB), lambda i: (i * 0, i * 0, i)),
            pl.BlockSpec((64, _TB), lambda i: (i * 0, i)),
            pl.BlockSpec((32, 64, _TB), lambda i: (i * 0, i * 0, i)),
            pl.BlockSpec((32, _TB), lambda i: (i * 0, i)),
        ],
        out_specs=[
            pl.BlockSpec((_TP, _TB), lambda i: (i * 0, i)),
            pl.BlockSpec((8, _TB), lambda i: (i * 0, i)),
        ],
        out_shape=[
            jax.ShapeDtypeStruct((_TP, t), jnp.int32),
            jax.ShapeDtypeStruct((8, t), jnp.float32),
        ],
        compiler_params=pltpu.CompilerParams(
            dimension_semantics=("arbitrary",)),
    )(locs_t, rmask_t, semb_t, nemb_t, mask_t)
    return sel, rl


def kernel(sta_idx, nei_idx, voc_idx, sta_emb, nei_emb, voc_emb, random_masks,
           mask, main_locations):
    t = sta_idx.shape[0]
    n_nbr = nei_idx.shape[1]
    k_voc = voc_idx.shape[1]

    idx_all = jnp.concatenate(
        [sta_idx[:, None], nei_idx, voc_idx], axis=1).astype(jnp.int32)
    idx_flat = idx_all.reshape(t * (1 + n_nbr + k_voc))

    tab_pl = main_locations.T.astype(jnp.int32)
    planes = [tab_pl[p] for p in range(_TP)]

    gathered = _sc_gather(planes, idx_flat)
    locs = gathered.reshape(_TP, t, 1 + n_nbr + k_voc)
    locs_t = locs.transpose(2, 0, 1)

    rmask_t = random_masks.astype(jnp.int32).reshape(t, _H, _TP).transpose(1, 2, 0)
    semb_t = sta_emb.T
    nemb_t = nei_emb.transpose(1, 2, 0)
    mask_t = mask.T

    sel, rl = _tc_compute(locs_t, rmask_t, semb_t, nemb_t, mask_t)

    selected_locs = sel.T.astype(jnp.int64)
    return selected_locs, rl[0], rl[1], rl[2]
